# Initial kernel scaffold; baseline (speedup 1.0000x reference)
import jax
import jax.numpy as jnp
from jax.experimental import pallas as pl

_VOCABS = [9, 16, 7, 15, 6, 5, 2, 40]
_DIMS = [3, 5, 2, 5, 2, 2, 2, 13]
_OFFS = [0, 3, 8, 10, 15, 17, 19, 21]
_B = 4096


def _body(x_ref, e0, e1, e2, e3, e4, e5, e6, e7, w1_ref, b1_ref, w2_ref, b2_ref,
          out_ref):
    embs = [e0, e1, e2, e3, e4, e5, e6, e7]
    x = x_ref[...]
    w1 = w1_ref[...]
    h = jnp.dot(x[:, 8:14], w1[34:40, :], preferred_element_type=jnp.float32)
    h = h + b1_ref[...]
    for i in range(8):
        ci = x[:, i:i + 1]
        iota = jax.lax.broadcasted_iota(jnp.float32, (_B, _VOCABS[i]), 1)
        m = (ci == iota).astype(jnp.float32)
        fused = jnp.dot(embs[i][...], w1[_OFFS[i]:_OFFS[i] + _DIMS[i], :],
                        preferred_element_type=jnp.float32)
        h = h + jnp.dot(m, fused, preferred_element_type=jnp.float32)
    h = jnp.maximum(h, 0.0)
    z = jnp.dot(h, w2_ref[...], preferred_element_type=jnp.float32) + b2_ref[...]
    out_ref[...] = 1.0 / (1.0 + jnp.exp(-z))


def kernel(x, emb0, emb1, emb2, emb3, emb4, emb5, emb6, emb7, W1, b1, W2, b2):
    out = pl.pallas_call(
        _body,
        out_shape=jax.ShapeDtypeStruct((_B, 1), jnp.float32),
    )(x, emb0, emb1, emb2, emb3, emb4, emb5, emb6, emb7,
      W1, b1.reshape(1, 50), W2, b2.reshape(1, 1))
    return out


# speedup vs baseline: 4.1245x; 4.1245x over previous
import jax
import jax.numpy as jnp
from jax.experimental import pallas as pl

_VOCABS = [9, 16, 7, 15, 6, 5, 2, 40]
_DIMS = [3, 5, 2, 5, 2, 2, 2, 13]
_OFFS = [0, 3, 8, 10, 15, 17, 19, 21]
_B = 4096


def _body(x_ref, e0, e1, e2, e3, e4, e5, e6, e7, w1_ref, b1_ref, w2_ref, b2_ref,
          out_ref):
    embs = [e0, e1, e2, e3, e4, e5, e6, e7]
    x = x_ref[...]
    w1 = w1_ref[...]
    h = jnp.dot(x[:, 8:14], w1[34:40, :], preferred_element_type=jnp.float32)
    h = h + b1_ref[...]
    for i in range(8):
        ci = x[:, i:i + 1].astype(jnp.int32)
        iota = jax.lax.broadcasted_iota(jnp.int32, (_B, _VOCABS[i]), 1)
        m = (ci == iota).astype(jnp.float32)
        fused = jnp.dot(embs[i][...], w1[_OFFS[i]:_OFFS[i] + _DIMS[i], :],
                        preferred_element_type=jnp.float32)
        h = h + jnp.dot(m, fused, preferred_element_type=jnp.float32)
    h = jnp.maximum(h, 0.0)
    z = jnp.dot(h, w2_ref[...], preferred_element_type=jnp.float32) + b2_ref[...]
    out_ref[...] = 1.0 / (1.0 + jnp.exp(-z))


def kernel(x, emb0, emb1, emb2, emb3, emb4, emb5, emb6, emb7, W1, b1, W2, b2):
    out = pl.pallas_call(
        _body,
        out_shape=jax.ShapeDtypeStruct((_B, 1), jnp.float32),
    )(x, emb0, emb1, emb2, emb3, emb4, emb5, emb6, emb7,
      W1, b1.reshape(1, 50), W2, b2.reshape(1, 1))
    return out
